# Initial kernel scaffold; baseline (speedup 1.0000x reference)
import jax
import jax.numpy as jnp
from jax.experimental import pallas as pl


def kernel(x, edge_index, batch, protein, W_rel0, b_rel0, W_root0, W_rel, b_rel, W_root, W_fc, b_fc, gamma, beta, W_fc2, b_fc2):
    raise NotImplementedError("write your pallas kernel here")


# speedup vs baseline: 1.9510x; 1.9510x over previous
import functools

import jax
import jax.numpy as jnp
from jax import lax
from jax.experimental import pallas as pl
from jax.experimental.pallas import tpu as pltpu
from jax.experimental.pallas import tpu_sc as plsc

_N = 10000
_E = 320000
_D = 128
_B = 256

_NC = 2
_NS = 16
_NW = _NC * _NS

_ROWS = 2528
_EPAD = _ROWS * _D
_BROWS = 32
_NBLK = _ROWS // _BROWS

_OWN = 320
_ACC = _NS * _OWN + 8
_TRASH = _NS * _OWN
_NSPEC = _EPAD - _E

_RING = 8192
_RINGR = _RING // _D
_CAPR = 2560
_GRP = 1024


def _mesh():
    return plsc.VectorSubcoreMesh(core_axis_name="c", subcore_axis_name="s")


def _bucket_body(src_hbm, dst_hbm, srcl_hbm, dstl_hbm, gcnt_hbm,
                 blk_s, blk_d, rsrc, rdst, cntb):
    c = lax.axis_index("c")
    s = lax.axis_index("s")
    w = c * _NS + s
    lo = w * _OWN
    hi = lo + _OWN
    cbase = c * (_NS * _OWN)
    lbase = w * _CAPR

    def flush(f):
        row = pl.multiple_of((f >> 7) & (_RINGR - 1), 8)
        grow = pl.multiple_of(lbase + (f >> 7), 8)
        pltpu.sync_copy(rsrc.at[pl.ds(row, 8)], srcl_hbm.at[pl.ds(grow, 8)])
        pltpu.sync_copy(rdst.at[pl.ds(row, 8)], dstl_hbm.at[pl.ds(grow, 8)])

    lane16 = lax.broadcasted_iota(jnp.int32, (16,), 0)
    dnums = lax.GatherDimensionNumbers(
        offset_dims=(), collapsed_slice_dims=(0,), start_index_map=(0,))

    def prefix16(x):
        for sh in (1, 2, 4, 8):
            idx = jnp.maximum(lane16 - sh, 0).reshape(16, 1)
            g = lax.gather(x, idx, dnums, (1,),
                           mode=lax.GatherScatterMode.PROMISE_IN_BOUNDS)
            x = x + jnp.where(lane16 >= sh, g, 0)
        return x

    def vreg(j, cnt):
        r = j >> 3
        k = (j & 7) * 16
        d = blk_d[r, pl.ds(k, 16)]
        sv = blk_s[r, pl.ds(k, 16)]
        m = jnp.logical_or(jnp.logical_and(d >= lo, d < hi), d < 0)
        cs = prefix16(jnp.where(m, 1, 0))
        pos = (cnt + cs - 1) & (_RING - 1)
        prow = pos >> 7
        pcol = pos & (_D - 1)
        dl = jnp.where(d < 0, _TRASH, d - cbase)
        plsc.store_scatter(rsrc, [prow, pcol], sv, mask=m)
        plsc.store_scatter(rdst, [prow, pcol], dl, mask=m)
        return cnt + cs[15]

    def block(b, carry):
        cnt, f = carry
        bo = pl.multiple_of(b * _BROWS, 8)
        pltpu.sync_copy(src_hbm.at[pl.ds(bo, _BROWS)], blk_s)
        pltpu.sync_copy(dst_hbm.at[pl.ds(bo, _BROWS)], blk_d)
        cnt = lax.fori_loop(0, _BROWS * 8, vreg, cnt)

        def fl(_, f):
            flush(f)
            return f + _GRP

        nfl = (cnt - f) >> 10
        f = lax.fori_loop(0, nfl, fl, f)
        return cnt, f

    cnt, f = lax.fori_loop(0, _NBLK, block, (0, 0))

    g = (cnt - _NSPEC + _GRP - 1) >> 10
    cntb[0, pl.ds(0, 16)] = jnp.full((16,), g, jnp.int32)
    pltpu.sync_copy(cntb, gcnt_hbm.at[pl.ds(pl.multiple_of(w * 8, 8), 8)])


@functools.cache
def _build_bucket():
    return pl.kernel(
        _bucket_body,
        out_type=[
            jax.ShapeDtypeStruct((_NW * _CAPR, _D), jnp.int32),
            jax.ShapeDtypeStruct((_NW * _CAPR, _D), jnp.int32),
            jax.ShapeDtypeStruct((_NW * 8, _D), jnp.int32),
        ],
        mesh=_mesh(),
        compiler_params=pltpu.CompilerParams(needs_layout_passes=False),
        scratch_types=[
            pltpu.VMEM((_BROWS, _D), jnp.int32),
            pltpu.VMEM((_BROWS, _D), jnp.int32),
            pltpu.VMEM((_RINGR, _D), jnp.int32),
            pltpu.VMEM((_RINGR, _D), jnp.int32),
            pltpu.VMEM((8, _D), jnp.int32),
        ],
    )


def _seg_body(h_hbm, srcl_hbm, dstl_hbm, gcnt_hbm, zeros_hbm, out_hbm,
              acc, src_v, dst_v, rows_v, cnt_v, sem):
    c = lax.axis_index("c")
    s = lax.axis_index("s")
    w = c * _NS + s
    lbase = w * _CAPR
    pltpu.sync_copy(zeros_hbm, acc.at[pl.ds(pl.multiple_of(s * _OWN, 8), _OWN)])
    pltpu.sync_copy(gcnt_hbm.at[pl.ds(pl.multiple_of(w * 8, 8), 8)], cnt_v)
    g = cnt_v[0, pl.ds(0, 16)][0]

    def grp(gi, carry):
        go = pl.multiple_of(lbase + gi * 8, 8)
        pltpu.sync_copy(srcl_hbm.at[pl.ds(go, 8)], src_v)
        pltpu.sync_copy(dstl_hbm.at[pl.ds(go, 8)], dst_v)
        for r in range(8):
            pltpu.async_copy(h_hbm.at[src_v.at[r]], rows_v, sem).wait()
            pltpu.sync_copy(rows_v, acc.at[dst_v.at[r]], add=True)
        return carry

    lax.fori_loop(0, g, grp, 0)
    pltpu.sync_copy(acc.at[pl.ds(pl.multiple_of(s * _OWN, 8), _OWN)],
                    out_hbm.at[pl.ds(pl.multiple_of(w * _OWN, 8), _OWN)])


@functools.cache
def _build_seg():
    return pl.kernel(
        _seg_body,
        out_type=jax.ShapeDtypeStruct((_NW * _OWN, _D), jnp.float32),
        mesh=_mesh(),
        compiler_params=pltpu.CompilerParams(needs_layout_passes=False),
        scratch_types=[
            pltpu.VMEM_SHARED((_ACC, _D), jnp.float32),
            pltpu.VMEM((8, _D), jnp.int32),
            pltpu.VMEM((8, _D), jnp.int32),
            pltpu.VMEM((_D, _D), jnp.float32),
            pltpu.VMEM((8, _D), jnp.int32),
            pltpu.SemaphoreType.DMA,
        ],
    )


def _layer_body(p_ref, h_ref, wr_ref, b_ref, wt_ref, o_ref):
    z = (jnp.dot(p_ref[...], wr_ref[...], preferred_element_type=jnp.float32)
         + b_ref[...]
         + jnp.dot(h_ref[...], wt_ref[...], preferred_element_type=jnp.float32))
    o_ref[...] = jnp.maximum(z, 0.0)


_BLK = 2000


def _layer(p, h, wr, b2, wt):
    return pl.pallas_call(
        _layer_body,
        grid=(_N // _BLK,),
        in_specs=[
            pl.BlockSpec((_BLK, _D), lambda i: (i, 0)),
            pl.BlockSpec((_BLK, _D), lambda i: (i, 0)),
            pl.BlockSpec((_D, _D), lambda i: (0, 0)),
            pl.BlockSpec((1, _D), lambda i: (0, 0)),
            pl.BlockSpec((_D, _D), lambda i: (0, 0)),
        ],
        out_specs=pl.BlockSpec((_BLK, _D), lambda i: (i, 0)),
        out_shape=jax.ShapeDtypeStruct((_N, _D), jnp.float32),
    )(p, h, wr, b2, wt)


def _head_body(h_ref, batch_ref, prot_ref, wfa_ref, wfb_ref, bfc_ref,
               gamma_ref, beta_ref, wf2_ref, bf2_ref, o_ref):
    ids = lax.broadcasted_iota(jnp.int32, (_B, _N), 0)
    oh = (ids == batch_ref[...]).astype(jnp.float32)
    h = h_ref[...]
    h1 = h.astype(jnp.bfloat16).astype(jnp.float32)
    r1 = h - h1
    h2 = r1.astype(jnp.bfloat16).astype(jnp.float32)
    h3 = r1 - h2
    sums = (jnp.dot(oh, h1, preferred_element_type=jnp.float32)
            + jnp.dot(oh, h2, preferred_element_type=jnp.float32)
            + jnp.dot(oh, h3, preferred_element_type=jnp.float32))
    counts = jnp.sum(oh, axis=1, keepdims=True)
    pooled = sums / jnp.maximum(counts, 1.0)
    z = (jnp.dot(pooled, wfa_ref[...], preferred_element_type=jnp.float32)
         + jnp.dot(prot_ref[...], wfb_ref[...], preferred_element_type=jnp.float32)
         + bfc_ref[...])
    mean = jnp.mean(z, axis=0, keepdims=True)
    var = jnp.mean((z - mean) ** 2, axis=0, keepdims=True)
    zn = (z - mean) * lax.rsqrt(var + 1e-5) * gamma_ref[...] + beta_ref[...]
    ze = jnp.where(zn > 0, zn, jnp.exp(zn) - 1.0)
    o_ref[...] = jnp.dot(ze, wf2_ref[...], preferred_element_type=jnp.float32) + bf2_ref[...]


def _head(h, batch2, prot8, wfa, wfb, bfc, gamma2, beta2, wf2, bf2):
    return pl.pallas_call(
        _head_body,
        out_shape=jax.ShapeDtypeStruct((_B, 8), jnp.float32),
    )(h, batch2, prot8, wfa, wfb, bfc, gamma2, beta2, wf2, bf2)


def kernel(x, edge_index, batch, protein, W_rel0, b_rel0, W_root0,
           W_rel, b_rel, W_root, W_fc, b_fc, gamma, beta, W_fc2, b_fc2):
    src = edge_index[0]
    dst = edge_index[1]
    padn = _EPAD - _E
    srcp = jnp.concatenate([src, jnp.zeros((padn,), jnp.int32)]).reshape(_ROWS, _D)
    dstp = jnp.concatenate([dst, jnp.full((padn,), -1, jnp.int32)]).reshape(_ROWS, _D)
    zeros_blk = jnp.zeros((_OWN, _D), jnp.float32)

    srcl, dstl, gcnt = _build_bucket()(srcp, dstp)

    h = jnp.pad(x, ((0, 0), (0, _D - 3)))
    wrs = [jnp.pad(W_rel0, ((0, _D - 3), (0, 0))), W_rel[0], W_rel[1], W_rel[2]]
    wts = [jnp.pad(W_root0, ((0, _D - 3), (0, 0))), W_root[0], W_root[1], W_root[2]]
    bs = [b_rel0.reshape(1, _D), b_rel[0].reshape(1, _D),
          b_rel[1].reshape(1, _D), b_rel[2].reshape(1, _D)]

    for l in range(4):
        agg = _build_seg()(h, srcl, dstl, gcnt, zeros_blk)
        h = _layer(agg[:_N], h, wrs[l], bs[l], wts[l])

    out8 = _head(
        h,
        batch.reshape(1, _N),
        jnp.pad(protein, ((0, 0), (0, 5))),
        W_fc[:_D],
        jnp.pad(W_fc[_D:], ((0, 5), (0, 0))),
        b_fc.reshape(1, 64),
        gamma.reshape(1, 64),
        beta.reshape(1, 64),
        jnp.pad(W_fc2, ((0, 0), (0, 7))),
        jnp.pad(b_fc2, (0, 7)).reshape(1, 8),
    )
    return out8[:, :1]


# speedup vs baseline: 2.0458x; 1.0486x over previous
import functools

import jax
import jax.numpy as jnp
from jax import lax
from jax.experimental import pallas as pl
from jax.experimental.pallas import tpu as pltpu
from jax.experimental.pallas import tpu_sc as plsc

_N = 10000
_E = 320000
_D = 128
_B = 256

_NC = 2
_NS = 16
_NW = _NC * _NS

_ROWS = 2528
_EPAD = _ROWS * _D
_BROWS = 32
_NBLK = _ROWS // _BROWS

_OWN = 320
_ACC = _NS * _OWN + 8
_TRASH = _NS * _OWN
_NSPEC = _EPAD - _E

_RING = 8192
_RINGR = _RING // _D
_CAPR = 2560
_GRP = 1024


def _mesh():
    return plsc.VectorSubcoreMesh(core_axis_name="c", subcore_axis_name="s")


def _bucket_body(src_hbm, dst_hbm, srcl_hbm, dstl_hbm, gcnt_hbm,
                 blk_s, blk_d, rsrc, rdst, cntb):
    c = lax.axis_index("c")
    s = lax.axis_index("s")
    w = c * _NS + s
    lo = w * _OWN
    hi = lo + _OWN
    cbase = c * (_NS * _OWN)
    lbase = w * _CAPR

    def flush(f):
        row = pl.multiple_of((f >> 7) & (_RINGR - 1), 8)
        grow = pl.multiple_of(lbase + (f >> 7), 8)
        pltpu.sync_copy(rsrc.at[pl.ds(row, 8)], srcl_hbm.at[pl.ds(grow, 8)])
        pltpu.sync_copy(rdst.at[pl.ds(row, 8)], dstl_hbm.at[pl.ds(grow, 8)])

    lane16 = lax.broadcasted_iota(jnp.int32, (16,), 0)
    dnums = lax.GatherDimensionNumbers(
        offset_dims=(), collapsed_slice_dims=(0,), start_index_map=(0,))

    def prefix16(x):
        for sh in (1, 2, 4, 8):
            idx = jnp.maximum(lane16 - sh, 0).reshape(16, 1)
            g = lax.gather(x, idx, dnums, (1,),
                           mode=lax.GatherScatterMode.PROMISE_IN_BOUNDS)
            x = x + jnp.where(lane16 >= sh, g, 0)
        return x

    def vreg(j, cnt):
        r = j >> 3
        k = (j & 7) * 16
        d = blk_d[r, pl.ds(k, 16)]
        sv = blk_s[r, pl.ds(k, 16)]
        m = jnp.logical_or(jnp.logical_and(d >= lo, d < hi), d < 0)
        cs = prefix16(jnp.where(m, 1, 0))
        pos = (cnt + cs - 1) & (_RING - 1)
        prow = pos >> 7
        pcol = pos & (_D - 1)
        dl = jnp.where(d < 0, _TRASH, d - cbase)
        plsc.store_scatter(rsrc, [prow, pcol], sv, mask=m)
        plsc.store_scatter(rdst, [prow, pcol], dl, mask=m)
        return cnt + cs[15]

    def block(b, carry):
        cnt, f = carry
        bo = pl.multiple_of(b * _BROWS, 8)
        pltpu.sync_copy(src_hbm.at[pl.ds(bo, _BROWS)], blk_s)
        pltpu.sync_copy(dst_hbm.at[pl.ds(bo, _BROWS)], blk_d)
        cnt = lax.fori_loop(0, _BROWS * 8, vreg, cnt)

        def fl(_, f):
            flush(f)
            return f + _GRP

        nfl = (cnt - f) >> 10
        f = lax.fori_loop(0, nfl, fl, f)
        return cnt, f

    cnt, f = lax.fori_loop(0, _NBLK, block, (0, 0))

    g = (cnt - _NSPEC + _GRP - 1) >> 10
    cntb[0, pl.ds(0, 16)] = jnp.full((16,), g, jnp.int32)
    pltpu.sync_copy(cntb, gcnt_hbm.at[pl.ds(pl.multiple_of(w * 8, 8), 8)])


@functools.cache
def _build_bucket():
    return pl.kernel(
        _bucket_body,
        out_type=[
            jax.ShapeDtypeStruct((_NW * _CAPR, _D), jnp.int32),
            jax.ShapeDtypeStruct((_NW * _CAPR, _D), jnp.int32),
            jax.ShapeDtypeStruct((_NW * 8, _D), jnp.int32),
        ],
        mesh=_mesh(),
        compiler_params=pltpu.CompilerParams(needs_layout_passes=False),
        scratch_types=[
            pltpu.VMEM((_BROWS, _D), jnp.int32),
            pltpu.VMEM((_BROWS, _D), jnp.int32),
            pltpu.VMEM((_RINGR, _D), jnp.int32),
            pltpu.VMEM((_RINGR, _D), jnp.int32),
            pltpu.VMEM((8, _D), jnp.int32),
        ],
    )


def _seg_body(h_hbm, srcl_hbm, dstl_hbm, gcnt_hbm, zeros_hbm, out_hbm,
              acc, src_v, dst_v, rows_v, cnt_v, sem, sem2):
    c = lax.axis_index("c")
    s = lax.axis_index("s")
    w = c * _NS + s
    lbase = w * _CAPR
    pltpu.sync_copy(zeros_hbm, acc.at[pl.ds(pl.multiple_of(s * _OWN, 8), _OWN)])
    pltpu.sync_copy(gcnt_hbm.at[pl.ds(pl.multiple_of(w * 8, 8), 8)], cnt_v)
    g = cnt_v[0, pl.ds(0, 16)][0]

    def grp(gi, carry):
        go = pl.multiple_of(lbase + gi * 8, 8)
        pltpu.sync_copy(srcl_hbm.at[pl.ds(go, 8)], src_v)
        pltpu.sync_copy(dstl_hbm.at[pl.ds(go, 8)], dst_v)
        cps = [None, None]
        cps[0] = pltpu.async_copy(h_hbm.at[src_v.at[0]], rows_v.at[0], sem)
        for r in range(8):
            cps[r % 2].wait()
            if r < 7:
                cps[(r + 1) % 2] = pltpu.async_copy(
                    h_hbm.at[src_v.at[r + 1]], rows_v.at[(r + 1) % 2], sem2)
            pltpu.sync_copy(rows_v.at[r % 2], acc.at[dst_v.at[r]], add=True)
        return carry

    lax.fori_loop(0, g, grp, 0)
    pltpu.sync_copy(acc.at[pl.ds(pl.multiple_of(s * _OWN, 8), _OWN)],
                    out_hbm.at[pl.ds(pl.multiple_of(w * _OWN, 8), _OWN)])


@functools.cache
def _build_seg():
    return pl.kernel(
        _seg_body,
        out_type=jax.ShapeDtypeStruct((_NW * _OWN, _D), jnp.float32),
        mesh=_mesh(),
        compiler_params=pltpu.CompilerParams(needs_layout_passes=False),
        scratch_types=[
            pltpu.VMEM_SHARED((_ACC, _D), jnp.float32),
            pltpu.VMEM((8, _D), jnp.int32),
            pltpu.VMEM((8, _D), jnp.int32),
            pltpu.VMEM((2, _D, _D), jnp.float32),
            pltpu.VMEM((8, _D), jnp.int32),
            pltpu.SemaphoreType.DMA,
            pltpu.SemaphoreType.DMA,
        ],
    )


def _layer_body(p_ref, h_ref, wr_ref, b_ref, wt_ref, o_ref):
    z = (jnp.dot(p_ref[...], wr_ref[...], preferred_element_type=jnp.float32)
         + b_ref[...]
         + jnp.dot(h_ref[...], wt_ref[...], preferred_element_type=jnp.float32))
    o_ref[...] = jnp.maximum(z, 0.0)


_BLK = 2000


def _layer(p, h, wr, b2, wt):
    return pl.pallas_call(
        _layer_body,
        grid=(_N // _BLK,),
        in_specs=[
            pl.BlockSpec((_BLK, _D), lambda i: (i, 0)),
            pl.BlockSpec((_BLK, _D), lambda i: (i, 0)),
            pl.BlockSpec((_D, _D), lambda i: (0, 0)),
            pl.BlockSpec((1, _D), lambda i: (0, 0)),
            pl.BlockSpec((_D, _D), lambda i: (0, 0)),
        ],
        out_specs=pl.BlockSpec((_BLK, _D), lambda i: (i, 0)),
        out_shape=jax.ShapeDtypeStruct((_N, _D), jnp.float32),
    )(p, h, wr, b2, wt)


def _head_body(h_ref, batch_ref, prot_ref, wfa_ref, wfb_ref, bfc_ref,
               gamma_ref, beta_ref, wf2_ref, bf2_ref, o_ref):
    ids = lax.broadcasted_iota(jnp.int32, (_B, _N), 0)
    oh = (ids == batch_ref[...]).astype(jnp.float32)
    h = h_ref[...]
    h1 = h.astype(jnp.bfloat16).astype(jnp.float32)
    r1 = h - h1
    h2 = r1.astype(jnp.bfloat16).astype(jnp.float32)
    h3 = r1 - h2
    sums = (jnp.dot(oh, h1, preferred_element_type=jnp.float32)
            + jnp.dot(oh, h2, preferred_element_type=jnp.float32)
            + jnp.dot(oh, h3, preferred_element_type=jnp.float32))
    counts = jnp.sum(oh, axis=1, keepdims=True)
    pooled = sums / jnp.maximum(counts, 1.0)
    z = (jnp.dot(pooled, wfa_ref[...], preferred_element_type=jnp.float32)
         + jnp.dot(prot_ref[...], wfb_ref[...], preferred_element_type=jnp.float32)
         + bfc_ref[...])
    mean = jnp.mean(z, axis=0, keepdims=True)
    var = jnp.mean((z - mean) ** 2, axis=0, keepdims=True)
    zn = (z - mean) * lax.rsqrt(var + 1e-5) * gamma_ref[...] + beta_ref[...]
    ze = jnp.where(zn > 0, zn, jnp.exp(zn) - 1.0)
    o_ref[...] = jnp.dot(ze, wf2_ref[...], preferred_element_type=jnp.float32) + bf2_ref[...]


def _head(h, batch2, prot8, wfa, wfb, bfc, gamma2, beta2, wf2, bf2):
    return pl.pallas_call(
        _head_body,
        out_shape=jax.ShapeDtypeStruct((_B, 8), jnp.float32),
    )(h, batch2, prot8, wfa, wfb, bfc, gamma2, beta2, wf2, bf2)


def kernel(x, edge_index, batch, protein, W_rel0, b_rel0, W_root0,
           W_rel, b_rel, W_root, W_fc, b_fc, gamma, beta, W_fc2, b_fc2):
    src = edge_index[0]
    dst = edge_index[1]
    padn = _EPAD - _E
    srcp = jnp.concatenate([src, jnp.zeros((padn,), jnp.int32)]).reshape(_ROWS, _D)
    dstp = jnp.concatenate([dst, jnp.full((padn,), -1, jnp.int32)]).reshape(_ROWS, _D)
    zeros_blk = jnp.zeros((_OWN, _D), jnp.float32)

    srcl, dstl, gcnt = _build_bucket()(srcp, dstp)

    h = jnp.pad(x, ((0, 0), (0, _D - 3)))
    wrs = [jnp.pad(W_rel0, ((0, _D - 3), (0, 0))), W_rel[0], W_rel[1], W_rel[2]]
    wts = [jnp.pad(W_root0, ((0, _D - 3), (0, 0))), W_root[0], W_root[1], W_root[2]]
    bs = [b_rel0.reshape(1, _D), b_rel[0].reshape(1, _D),
          b_rel[1].reshape(1, _D), b_rel[2].reshape(1, _D)]

    for l in range(4):
        agg = _build_seg()(h, srcl, dstl, gcnt, zeros_blk)
        h = _layer(agg[:_N], h, wrs[l], bs[l], wts[l])

    out8 = _head(
        h,
        batch.reshape(1, _N),
        jnp.pad(protein, ((0, 0), (0, 5))),
        W_fc[:_D],
        jnp.pad(W_fc[_D:], ((0, 5), (0, 0))),
        b_fc.reshape(1, 64),
        gamma.reshape(1, 64),
        beta.reshape(1, 64),
        jnp.pad(W_fc2, ((0, 0), (0, 7))),
        jnp.pad(b_fc2, (0, 7)).reshape(1, 8),
    )
    return out8[:, :1]
